# baseline (speedup 1.0000x reference)
import functools

import jax
import jax.numpy as jnp
from jax import lax
from jax.experimental import pallas as pl
from jax.experimental.pallas import tpu as pltpu
from jax.experimental.pallas import tpu_sc as plsc

N = 26128
E = 418048
D = 128
REL = 4
NIDX = 4096

NC = 2
NS = 16
BLK = 128
CHK = 26
NCHK = 8
NBLK = CHK * NCHK
EPT = E // NS
EPT_PAD = NBLK * BLK
NPT = 1632
NREM = N - NS * NPT
DUMMY = N

EPW = E // (NC * NS)
NCHK_D = 7
NBLK_D = CHK * NCHK_D
EPW_PAD = NBLK_D * BLK

def _sc_mesh():
    return plsc.VectorSubcoreMesh(
        core_axis_name="c", subcore_axis_name="s",
        num_cores=NC, num_subcores=NS)


_SC_PARAMS = pltpu.CompilerParams(use_tc_tiling_on_sc=False)


@functools.cache
def _get_sc_deg():
  return functools.partial(
    pl.kernel,
    out_type=jax.ShapeDtypeStruct((NC, N, 16), jnp.float32),
    mesh=_sc_mesh(),
    scratch_types=[
        pltpu.VMEM((CHK, BLK), jnp.int32),
        pltpu.VMEM((BLK, 16), jnp.float32),
        pltpu.VMEM_SHARED((N + 16, 16), jnp.float32),
        pltpu.SemaphoreType.DMA,
    ],
    compiler_params=_SC_PARAMS,
  )(_sc_deg)


def _sc_deg(dslab_hbm, zeros_hbm, ones_hbm, out_hbm, idx_v, ones_v, acc, sem):
    c = lax.axis_index("c")
    s = lax.axis_index("s")
    pltpu.sync_copy(zeros_hbm, acc.at[pl.ds(s * NPT, NPT)])

    @pl.when(s == NS - 1)
    def _zr():
        pltpu.sync_copy(zeros_hbm.at[pl.ds(0, NREM)],
                        acc.at[pl.ds(NS * NPT, NREM)])

    pltpu.sync_copy(ones_hbm, ones_v)
    plsc.subcore_barrier()
    w = c * NS + s

    def chunk(q, carry):
        pltpu.sync_copy(dslab_hbm.at[w].at[pl.ds(q * CHK, CHK)], idx_v)

        def body(j, carry2):
            pltpu.sync_copy(ones_v, acc.at[idx_v.at[j]], add=True)
            return carry2

        lax.fori_loop(0, CHK, body, 0)
        return carry

    lax.fori_loop(0, NCHK_D, chunk, 0)
    plsc.subcore_barrier()
    pltpu.sync_copy(acc.at[pl.ds(s * NPT, NPT)],
                    out_hbm.at[c].at[pl.ds(s * NPT, NPT)])

    @pl.when(s == NS - 1)
    def _fr():
        pltpu.sync_copy(acc.at[pl.ds(NS * NPT, NREM)],
                        out_hbm.at[c].at[pl.ds(NS * NPT, NREM)])


@functools.cache
def _get_sc_edge():
  return functools.partial(
    pl.kernel,
    out_type=jax.ShapeDtypeStruct((6, N, 64), jnp.float32),
    mesh=_sc_mesh(),
    scratch_types=[
        pltpu.VMEM((CHK, BLK), jnp.int32),
        pltpu.VMEM((CHK, BLK), jnp.int32),
        pltpu.VMEM((BLK, 64), jnp.float32),
        pltpu.VMEM((BLK, 64), jnp.float32),
        pltpu.VMEM_SHARED((N + 16, 64), jnp.float32),
        pltpu.SemaphoreType.DMA,
        pltpu.SemaphoreType.DMA,
    ],
    compiler_params=_SC_PARAMS,
  )(_sc_edge)


def _sc_edge(tbl_hbm, tu_hbm, gc_hbm, sslab_hbm, zeros_hbm, out_hbm,
             idxg_v, idxs_v, gbuf0, gbuf1, acc, sem0, sem1):
    c = lax.axis_index("c")
    s = lax.axis_index("s")

    def one_round(base_hbm, w, out_slot):
        pltpu.sync_copy(zeros_hbm, acc.at[pl.ds(s * NPT, NPT)])

        @pl.when(s == NS - 1)
        def _zr():
            pltpu.sync_copy(zeros_hbm.at[pl.ds(0, NREM)],
                            acc.at[pl.ds(NS * NPT, NREM)])

        plsc.subcore_barrier()

        def chunk(q, carry):
            d1 = pltpu.async_copy(base_hbm.at[w].at[pl.ds(q * CHK, CHK)],
                                  idxg_v, sem0)
            d2 = pltpu.async_copy(sslab_hbm.at[s].at[pl.ds(q * CHK, CHK)],
                                  idxs_v, sem1)
            d1.wait()
            d2.wait()

            bufs = (gbuf0, gbuf1)
            gsems = (sem0, sem1)
            gd = [pltpu.async_copy(tbl_hbm.at[idxg_v.at[0]], gbuf0, sem0),
                  pltpu.async_copy(tbl_hbm.at[idxg_v.at[1]], gbuf1, sem1)]
            for j in range(CHK):
                p = j % 2
                gd[p].wait()
                pltpu.sync_copy(bufs[p], acc.at[idxs_v.at[j]], add=True)
                if j + 2 < CHK:
                    gd[p] = pltpu.async_copy(
                        tbl_hbm.at[idxg_v.at[j + 2]], bufs[p], gsems[p])
            return carry

        lax.fori_loop(0, NCHK, chunk, 0)
        plsc.subcore_barrier()
        pltpu.sync_copy(acc.at[pl.ds(s * NPT, NPT)],
                        out_hbm.at[out_slot].at[pl.ds(s * NPT, NPT)])

        @pl.when(s == NS - 1)
        def _fr():
            pltpu.sync_copy(acc.at[pl.ds(NS * NPT, NREM)],
                            out_hbm.at[out_slot].at[pl.ds(NS * NPT, NREM)])

        plsc.subcore_barrier()

    for r in range(2):
        q = 2 * c + r
        one_round(tu_hbm, q * NS + s, q)
    one_round(gc_hbm, c * NS + s, 4 + c)


@functools.cache
def _get_sc_rowgather():
  return functools.partial(
    pl.kernel,
    out_type=jax.ShapeDtypeStruct((NIDX, D), jnp.float32),
    mesh=_sc_mesh(),
    scratch_types=[
        pltpu.VMEM((1, BLK), jnp.int32),
        pltpu.VMEM((BLK, D), jnp.float32),
        pltpu.SemaphoreType.DMA,
    ],
    compiler_params=_SC_PARAMS,
  )(_sc_rowgather)


def _sc_rowgather(h_hbm, idx_hbm, out_hbm, idx_v, rows_v, sem):
    c = lax.axis_index("c")
    s = lax.axis_index("s")
    w = c * NS + s
    pltpu.sync_copy(idx_hbm.at[w], idx_v)
    pltpu.async_copy(h_hbm.at[idx_v.at[0]], rows_v, sem).wait()
    pltpu.sync_copy(rows_v, out_hbm.at[pl.ds(w * BLK, BLK)])


BN_B = 512
BN_G = (N + BN_B - 1) // BN_B


def _bn_mm_kernel(x_ref, w_ref, b_ref, y_ref, s_ref):
    i = pl.program_id(0)
    y = x_ref[...] @ w_ref[...] + b_ref[...]
    y_ref[...] = y
    rows = lax.broadcasted_iota(jnp.int32, y.shape, 0) + i * BN_B
    yv = jnp.where(rows < N, y, 0.0)

    @pl.when(i == 0)
    def _init():
        s_ref[...] = jnp.zeros_like(s_ref)

    s_ref[0:1, :] += jnp.sum(yv, axis=0, keepdims=True)
    s_ref[1:2, :] += jnp.sum(yv * yv, axis=0, keepdims=True)


def _emit_tables(x, dinv_ref, win_ref, bin_ref, wrel_ref, tbl_ref):
    hin = x @ win_ref[...] + bin_ref[...]
    es = []
    rs = []
    for r in range(REL):
        hc = hin @ wrel_ref[r]
        e = jnp.exp(hc)
        es.append(e)
        rs.append(hc * e)
    for q in range(4):
        lo, hi = 32 * q, 32 * q + 32
        for rr in range(2):
            tbl_ref[2 * q + rr, :, :] = jnp.concatenate(
                [es[2 * rr][:, lo:hi], rs[2 * rr][:, lo:hi],
                 es[2 * rr + 1][:, lo:hi], rs[2 * rr + 1][:, lo:hi]], axis=1)
    tbl_ref[8, :, :] = dinv_ref[...] * hin


def _tab0_kernel(y_ref, ab_ref, dinv_ref, win_ref, bin_ref, wrel_ref,
                 tbl_ref):
    x = jax.nn.relu(y_ref[...] * ab_ref[0:1, :] + ab_ref[1:2, :])
    _emit_tables(x, dinv_ref, win_ref, bin_ref, wrel_ref, tbl_ref)


def _combine(sc_ref, dinv_ref):
    t = jnp.concatenate([sc_ref[q, :, 0:32] for q in range(4)], axis=1)
    u = jnp.concatenate([sc_ref[q, :, 32:64] for q in range(4)], axis=1)
    g = jnp.concatenate([sc_ref[4], sc_ref[5]], axis=1)
    msg = u / (t + 1e-16)
    return dinv_ref[...] * g + 0.5 * jax.nn.relu(msg)


def _comb_tab_kernel(sc_ref, dinv_ref, wout_ref, bout_ref, win_ref, bin_ref,
                     wrel_ref, tbl_ref):
    z = _combine(sc_ref, dinv_ref)
    x = z @ wout_ref[...] + bout_ref[...]
    _emit_tables(x, dinv_ref, win_ref, bin_ref, wrel_ref, tbl_ref)


def _comb_final_kernel(sc_ref, dinv_ref, wout_ref, bout_ref, h_ref):
    z = _combine(sc_ref, dinv_ref)
    h = z @ wout_ref[...] + bout_ref[...]
    h_ref[...] = h * 0.5 * (1.0 + jax.lax.erf(h * (2.0 ** -0.5)))


def _row_spec(b, d):
    return pl.BlockSpec((b, d), lambda i: (i, 0))


def _const_spec(shape):
    nd = len(shape)
    return pl.BlockSpec(shape, lambda i, _n=nd: (0,) * _n)


def _pad_slab(a, nrows, nblk, fill):
    a = a.reshape(nrows, -1)
    pad = nblk * BLK - a.shape[1]
    a = jnp.pad(a, ((0, 0), (0, pad)), constant_values=fill)
    return a.reshape(nrows, nblk, BLK)


def kernel(x, W_proj, b_proj, gamma, beta, W_in0, b_in0, W_rel0, W_out0, b_out0,
           W_in1, b_in1, W_rel1, W_out1, b_out1, edge_index, idx, edge_type, edge_weight):
    f32 = jnp.float32
    src = edge_index[0]
    dst = edge_index[1]
    et = edge_type

    base2 = (et >> 1) * (2 * N) + 2 * dst + (et & 1)
    b2_slab = _pad_slab(base2, NS, NBLK, 0)
    g2_slab = _pad_slab(2 * dst, NS, NBLK, 0)
    s_slab = _pad_slab(src, NS, NBLK, DUMMY)
    offs = (jnp.arange(4, dtype=jnp.int32) * (4 * N))[:, None, None, None]
    tu_slab = (b2_slab[None] + offs).reshape(4 * NS, NBLK, BLK)
    goffs = (16 * N + jnp.arange(2, dtype=jnp.int32))[:, None, None, None]
    gc_slab = (g2_slab[None] + goffs).reshape(2 * NS, NBLK, BLK)
    d32_slab = _pad_slab(dst, NC * NS, NBLK_D, DUMMY)
    idx_slab = idx.astype(jnp.int32).reshape(NC * NS, 1, BLK)
    zeros64 = jnp.zeros((NPT, 64), f32)
    zeros16 = jnp.zeros((NPT, 16), f32)
    ones16 = jnp.ones((BLK, 16), f32)

    degp = _get_sc_deg()(d32_slab, zeros16, ones16)
    deg = degp[0, :, 0] + degp[1, :, 0]
    dinv = jnp.where(deg > 0, lax.rsqrt(deg), 0.0)[:, None]

    y, sums = pl.pallas_call(
        _bn_mm_kernel,
        grid=(BN_G,),
        in_specs=[_row_spec(BN_B, D), _const_spec((D, D)), _const_spec((1, D))],
        out_specs=[_row_spec(BN_B, D), _const_spec((2, D))],
        out_shape=[jax.ShapeDtypeStruct((N, D), f32),
                   jax.ShapeDtypeStruct((2, D), f32)],
    )(x, W_proj, b_proj.reshape(1, D))
    mean = sums[0] / N
    var = sums[1] / N - mean * mean
    a_scale = gamma * lax.rsqrt(var + 1e-5)
    ab = jnp.stack([a_scale, beta - mean * a_scale])

    tab_grid = dict(
        grid=(BN_G,),
        out_specs=pl.BlockSpec((9, BN_B, D), lambda i: (0, i, 0)),
        out_shape=jax.ShapeDtypeStruct((9, N, D), f32),
    )

    tbl1 = pl.pallas_call(
        _tab0_kernel,
        in_specs=[_row_spec(BN_B, D), _const_spec((2, D)), _row_spec(BN_B, 1),
                  _const_spec((D, D)), _const_spec((1, D)),
                  _const_spec((REL, D, D))],
        **tab_grid,
    )(y, ab, dinv, W_in0, b_in0.reshape(1, D), W_rel0)
    sc1 = _get_sc_edge()(tbl1.reshape(18 * N, 64), tu_slab, gc_slab, s_slab,
                         zeros64)

    tbl2 = pl.pallas_call(
        _comb_tab_kernel,
        in_specs=[pl.BlockSpec((6, BN_B, 64), lambda i: (0, i, 0)),
                  _row_spec(BN_B, 1), _const_spec((D, D)), _const_spec((1, D)),
                  _const_spec((D, D)), _const_spec((1, D)),
                  _const_spec((REL, D, D))],
        **tab_grid,
    )(sc1, dinv, W_out0, b_out0.reshape(1, D), W_in1, b_in1.reshape(1, D),
      W_rel1)
    sc2 = _get_sc_edge()(tbl2.reshape(18 * N, 64), tu_slab, gc_slab, s_slab,
                         zeros64)

    h2 = pl.pallas_call(
        _comb_final_kernel,
        grid=(BN_G,),
        in_specs=[pl.BlockSpec((6, BN_B, 64), lambda i: (0, i, 0)),
                  _row_spec(BN_B, 1), _const_spec((D, D)), _const_spec((1, D))],
        out_specs=_row_spec(BN_B, D),
        out_shape=jax.ShapeDtypeStruct((N, D), f32),
    )(sc2, dinv, W_out1, b_out1.reshape(1, D))

    return _get_sc_rowgather()(h2, idx_slab)


# speedup vs baseline: 1.0447x; 1.0447x over previous
import functools

import jax
import jax.numpy as jnp
from jax import lax
from jax.experimental import pallas as pl
from jax.experimental.pallas import tpu as pltpu
from jax.experimental.pallas import tpu_sc as plsc

N = 26128
E = 418048
D = 128
REL = 4
NIDX = 4096

NC = 2
NS = 16
BLK = 128
CHK = 16
NCHK = 13
NBLK = CHK * NCHK
EPT = E // NS
EPT_PAD = NBLK * BLK
NPT = 1632
NREM = N - NS * NPT
DUMMY = N

EPW = E // (NC * NS)
NCHK_D = 7
NBLK_D = CHK * NCHK_D
EPW_PAD = NBLK_D * BLK

def _sc_mesh():
    return plsc.VectorSubcoreMesh(
        core_axis_name="c", subcore_axis_name="s",
        num_cores=NC, num_subcores=NS)


_SC_PARAMS = pltpu.CompilerParams(use_tc_tiling_on_sc=False)


@functools.cache
def _get_sc_deg():
  return functools.partial(
    pl.kernel,
    out_type=jax.ShapeDtypeStruct((NC, N, 16), jnp.float32),
    mesh=_sc_mesh(),
    scratch_types=[
        pltpu.VMEM((CHK, BLK), jnp.int32),
        pltpu.VMEM((BLK, 16), jnp.float32),
        pltpu.VMEM_SHARED((N + 16, 16), jnp.float32),
        pltpu.SemaphoreType.DMA,
    ],
    compiler_params=_SC_PARAMS,
  )(_sc_deg)


def _sc_deg(dslab_hbm, zeros_hbm, ones_hbm, out_hbm, idx_v, ones_v, acc, sem):
    c = lax.axis_index("c")
    s = lax.axis_index("s")
    pltpu.sync_copy(zeros_hbm, acc.at[pl.ds(s * NPT, NPT)])

    @pl.when(s == NS - 1)
    def _zr():
        pltpu.sync_copy(zeros_hbm.at[pl.ds(0, NREM)],
                        acc.at[pl.ds(NS * NPT, NREM)])

    pltpu.sync_copy(ones_hbm, ones_v)
    plsc.subcore_barrier()
    w = c * NS + s

    def chunk(q, carry):
        pltpu.sync_copy(dslab_hbm.at[w].at[pl.ds(q * CHK, CHK)], idx_v)

        def body(j, carry2):
            pltpu.sync_copy(ones_v, acc.at[idx_v.at[j]], add=True)
            return carry2

        lax.fori_loop(0, CHK, body, 0)
        return carry

    lax.fori_loop(0, NCHK_D, chunk, 0)
    plsc.subcore_barrier()
    pltpu.sync_copy(acc.at[pl.ds(s * NPT, NPT)],
                    out_hbm.at[c].at[pl.ds(s * NPT, NPT)])

    @pl.when(s == NS - 1)
    def _fr():
        pltpu.sync_copy(acc.at[pl.ds(NS * NPT, NREM)],
                        out_hbm.at[c].at[pl.ds(NS * NPT, NREM)])


@functools.cache
def _get_sc_edge():
  return functools.partial(
    pl.kernel,
    out_type=jax.ShapeDtypeStruct((6, N, 64), jnp.float32),
    mesh=_sc_mesh(),
    scratch_types=[
        pltpu.VMEM((CHK, BLK), jnp.int32),
        pltpu.VMEM((CHK, BLK), jnp.int32),
        pltpu.VMEM((BLK, 64), jnp.float32),
        pltpu.VMEM((BLK, 64), jnp.float32),
        pltpu.VMEM_SHARED((N + 16, 64), jnp.float32),
        pltpu.SemaphoreType.DMA,
        pltpu.SemaphoreType.DMA,
    ],
    compiler_params=_SC_PARAMS,
  )(_sc_edge)


def _sc_edge(tbl_hbm, tu_hbm, gc_hbm, sslab_hbm, zeros_hbm, out_hbm,
             idxg_v, idxs_v, gbuf0, gbuf1, acc, sem0, sem1):
    c = lax.axis_index("c")
    s = lax.axis_index("s")

    def one_round(base_hbm, w, out_slot):
        pltpu.sync_copy(zeros_hbm, acc.at[pl.ds(s * NPT, NPT)])

        @pl.when(s == NS - 1)
        def _zr():
            pltpu.sync_copy(zeros_hbm.at[pl.ds(0, NREM)],
                            acc.at[pl.ds(NS * NPT, NREM)])

        plsc.subcore_barrier()

        def chunk(q, carry):
            d1 = pltpu.async_copy(base_hbm.at[w].at[pl.ds(q * CHK, CHK)],
                                  idxg_v, sem0)
            d2 = pltpu.async_copy(sslab_hbm.at[s].at[pl.ds(q * CHK, CHK)],
                                  idxs_v, sem1)
            d1.wait()
            d2.wait()

            bufs = (gbuf0, gbuf1)
            gsems = (sem0, sem1)
            gd = [pltpu.async_copy(tbl_hbm.at[idxg_v.at[0]], gbuf0, sem0),
                  pltpu.async_copy(tbl_hbm.at[idxg_v.at[1]], gbuf1, sem1)]
            for j in range(CHK):
                p = j % 2
                gd[p].wait()
                pltpu.sync_copy(bufs[p], acc.at[idxs_v.at[j]], add=True)
                if j + 2 < CHK:
                    gd[p] = pltpu.async_copy(
                        tbl_hbm.at[idxg_v.at[j + 2]], bufs[p], gsems[p])
            return carry

        lax.fori_loop(0, NCHK, chunk, 0)
        plsc.subcore_barrier()
        pltpu.sync_copy(acc.at[pl.ds(s * NPT, NPT)],
                        out_hbm.at[out_slot].at[pl.ds(s * NPT, NPT)])

        @pl.when(s == NS - 1)
        def _fr():
            pltpu.sync_copy(acc.at[pl.ds(NS * NPT, NREM)],
                            out_hbm.at[out_slot].at[pl.ds(NS * NPT, NREM)])

        plsc.subcore_barrier()

    for r in range(2):
        q = 2 * c + r
        one_round(tu_hbm, q * NS + s, q)
    one_round(gc_hbm, c * NS + s, 4 + c)


@functools.cache
def _get_sc_rowgather():
  return functools.partial(
    pl.kernel,
    out_type=jax.ShapeDtypeStruct((NIDX, D), jnp.float32),
    mesh=_sc_mesh(),
    scratch_types=[
        pltpu.VMEM((1, BLK), jnp.int32),
        pltpu.VMEM((BLK, D), jnp.float32),
        pltpu.SemaphoreType.DMA,
    ],
    compiler_params=_SC_PARAMS,
  )(_sc_rowgather)


def _sc_rowgather(h_hbm, idx_hbm, out_hbm, idx_v, rows_v, sem):
    c = lax.axis_index("c")
    s = lax.axis_index("s")
    w = c * NS + s
    pltpu.sync_copy(idx_hbm.at[w], idx_v)
    pltpu.async_copy(h_hbm.at[idx_v.at[0]], rows_v, sem).wait()
    pltpu.sync_copy(rows_v, out_hbm.at[pl.ds(w * BLK, BLK)])


BN_B = 512
BN_G = (N + BN_B - 1) // BN_B


def _bn_mm_kernel(x_ref, w_ref, b_ref, y_ref, s_ref):
    i = pl.program_id(0)
    y = x_ref[...] @ w_ref[...] + b_ref[...]
    y_ref[...] = y
    rows = lax.broadcasted_iota(jnp.int32, y.shape, 0) + i * BN_B
    yv = jnp.where(rows < N, y, 0.0)

    @pl.when(i == 0)
    def _init():
        s_ref[...] = jnp.zeros_like(s_ref)

    s_ref[0:1, :] += jnp.sum(yv, axis=0, keepdims=True)
    s_ref[1:2, :] += jnp.sum(yv * yv, axis=0, keepdims=True)


def _emit_tables(x, dinv_ref, win_ref, bin_ref, wrel_ref, tbl_ref):
    hin = x @ win_ref[...] + bin_ref[...]
    es = []
    rs = []
    for r in range(REL):
        hc = hin @ wrel_ref[r]
        e = jnp.exp(hc)
        es.append(e)
        rs.append(hc * e)
    for q in range(4):
        lo, hi = 32 * q, 32 * q + 32
        for rr in range(2):
            tbl_ref[2 * q + rr, :, :] = jnp.concatenate(
                [es[2 * rr][:, lo:hi], rs[2 * rr][:, lo:hi],
                 es[2 * rr + 1][:, lo:hi], rs[2 * rr + 1][:, lo:hi]], axis=1)
    tbl_ref[8, :, :] = dinv_ref[...] * hin


def _tab0_kernel(y_ref, ab_ref, dinv_ref, win_ref, bin_ref, wrel_ref,
                 tbl_ref):
    x = jax.nn.relu(y_ref[...] * ab_ref[0:1, :] + ab_ref[1:2, :])
    _emit_tables(x, dinv_ref, win_ref, bin_ref, wrel_ref, tbl_ref)


def _combine(sc_ref, dinv_ref):
    t = jnp.concatenate([sc_ref[q, :, 0:32] for q in range(4)], axis=1)
    u = jnp.concatenate([sc_ref[q, :, 32:64] for q in range(4)], axis=1)
    g = jnp.concatenate([sc_ref[4], sc_ref[5]], axis=1)
    msg = u / (t + 1e-16)
    return dinv_ref[...] * g + 0.5 * jax.nn.relu(msg)


def _comb_tab_kernel(sc_ref, dinv_ref, wout_ref, bout_ref, win_ref, bin_ref,
                     wrel_ref, tbl_ref):
    z = _combine(sc_ref, dinv_ref)
    x = z @ wout_ref[...] + bout_ref[...]
    _emit_tables(x, dinv_ref, win_ref, bin_ref, wrel_ref, tbl_ref)


def _comb_final_kernel(sc_ref, dinv_ref, wout_ref, bout_ref, h_ref):
    z = _combine(sc_ref, dinv_ref)
    h = z @ wout_ref[...] + bout_ref[...]
    h_ref[...] = h * 0.5 * (1.0 + jax.lax.erf(h * (2.0 ** -0.5)))


def _row_spec(b, d):
    return pl.BlockSpec((b, d), lambda i: (i, 0))


def _const_spec(shape):
    nd = len(shape)
    return pl.BlockSpec(shape, lambda i, _n=nd: (0,) * _n)


def _pad_slab(a, nrows, nblk, fill):
    a = a.reshape(nrows, -1)
    pad = nblk * BLK - a.shape[1]
    a = jnp.pad(a, ((0, 0), (0, pad)), constant_values=fill)
    return a.reshape(nrows, nblk, BLK)


def kernel(x, W_proj, b_proj, gamma, beta, W_in0, b_in0, W_rel0, W_out0, b_out0,
           W_in1, b_in1, W_rel1, W_out1, b_out1, edge_index, idx, edge_type, edge_weight):
    f32 = jnp.float32
    src = edge_index[0]
    dst = edge_index[1]
    et = edge_type

    base2 = (et >> 1) * (2 * N) + 2 * dst + (et & 1)
    b2_slab = _pad_slab(base2, NS, NBLK, 0)
    g2_slab = _pad_slab(2 * dst, NS, NBLK, 0)
    s_slab = _pad_slab(src, NS, NBLK, DUMMY)
    offs = (jnp.arange(4, dtype=jnp.int32) * (4 * N))[:, None, None, None]
    tu_slab = (b2_slab[None] + offs).reshape(4 * NS, NBLK, BLK)
    goffs = (16 * N + jnp.arange(2, dtype=jnp.int32))[:, None, None, None]
    gc_slab = (g2_slab[None] + goffs).reshape(2 * NS, NBLK, BLK)
    d32_slab = _pad_slab(dst, NC * NS, NBLK_D, DUMMY)
    idx_slab = idx.astype(jnp.int32).reshape(NC * NS, 1, BLK)
    zeros64 = jnp.zeros((NPT, 64), f32)
    zeros16 = jnp.zeros((NPT, 16), f32)
    ones16 = jnp.ones((BLK, 16), f32)

    degp = _get_sc_deg()(d32_slab, zeros16, ones16)
    deg = degp[0, :, 0] + degp[1, :, 0]
    dinv = jnp.where(deg > 0, lax.rsqrt(deg), 0.0)[:, None]

    y, sums = pl.pallas_call(
        _bn_mm_kernel,
        grid=(BN_G,),
        in_specs=[_row_spec(BN_B, D), _const_spec((D, D)), _const_spec((1, D))],
        out_specs=[_row_spec(BN_B, D), _const_spec((2, D))],
        out_shape=[jax.ShapeDtypeStruct((N, D), f32),
                   jax.ShapeDtypeStruct((2, D), f32)],
    )(x, W_proj, b_proj.reshape(1, D))
    mean = sums[0] / N
    var = sums[1] / N - mean * mean
    a_scale = gamma * lax.rsqrt(var + 1e-5)
    ab = jnp.stack([a_scale, beta - mean * a_scale])

    tab_grid = dict(
        grid=(BN_G,),
        out_specs=pl.BlockSpec((9, BN_B, D), lambda i: (0, i, 0)),
        out_shape=jax.ShapeDtypeStruct((9, N, D), f32),
    )

    tbl1 = pl.pallas_call(
        _tab0_kernel,
        in_specs=[_row_spec(BN_B, D), _const_spec((2, D)), _row_spec(BN_B, 1),
                  _const_spec((D, D)), _const_spec((1, D)),
                  _const_spec((REL, D, D))],
        **tab_grid,
    )(y, ab, dinv, W_in0, b_in0.reshape(1, D), W_rel0)
    sc1 = _get_sc_edge()(tbl1.reshape(18 * N, 64), tu_slab, gc_slab, s_slab,
                         zeros64)

    tbl2 = pl.pallas_call(
        _comb_tab_kernel,
        in_specs=[pl.BlockSpec((6, BN_B, 64), lambda i: (0, i, 0)),
                  _row_spec(BN_B, 1), _const_spec((D, D)), _const_spec((1, D)),
                  _const_spec((D, D)), _const_spec((1, D)),
                  _const_spec((REL, D, D))],
        **tab_grid,
    )(sc1, dinv, W_out0, b_out0.reshape(1, D), W_in1, b_in1.reshape(1, D),
      W_rel1)
    sc2 = _get_sc_edge()(tbl2.reshape(18 * N, 64), tu_slab, gc_slab, s_slab,
                         zeros64)

    h2 = pl.pallas_call(
        _comb_final_kernel,
        grid=(BN_G,),
        in_specs=[pl.BlockSpec((6, BN_B, 64), lambda i: (0, i, 0)),
                  _row_spec(BN_B, 1), _const_spec((D, D)), _const_spec((1, D))],
        out_specs=_row_spec(BN_B, D),
        out_shape=jax.ShapeDtypeStruct((N, D), f32),
    )(sc2, dinv, W_out1, b_out1.reshape(1, D))

    return _get_sc_rowgather()(h2, idx_slab)
